# Initial kernel scaffold; baseline (speedup 1.0000x reference)
import jax
import jax.numpy as jnp
from jax import lax
from jax.experimental import pallas as pl
from jax.experimental.pallas import tpu as pltpu
from jax.experimental.pallas import tpu_sc as plsc

HALF = 25000
EMB = 64
E = 800000
N_LAYERS = 3
NC = 2
NS = 16
L = 16

P = 25088
RPS = P // NS
DUMMY = HALF
CH = 128
E_PAD = 819200
ROWS = E_PAD // CH
RPSUB = ROWS // NS
SUPER = 16
NSUP = RPSUB // SUPER


def _layer_body(x_hbm, src_hbm, dst_hbm, w_hbm, z_hbm, out_hbm,
                acc, srcb, dstb, wb, dlb, rowsv, sem):
    c = lax.axis_index("c")
    s = lax.axis_index("s")

    pltpu.sync_copy(z_hbm.at[pl.ds(s * RPS, RPS), :],
                    acc.at[pl.ds(s * RPS, RPS), :])
    plsc.subcore_barrier()

    dst_base = c * HALF
    row0 = s * RPSUB

    def super_body(it, carry):
        r0 = row0 + it * SUPER
        pltpu.sync_copy(src_hbm.at[pl.ds(r0, SUPER), :], srcb)
        pltpu.sync_copy(dst_hbm.at[pl.ds(r0, SUPER), :], dstb)
        pltpu.sync_copy(w_hbm.at[pl.ds(r0, SUPER), :], wb)
        for j in range(SUPER):
            for g in range(8):
                sl = pl.ds(g * L, L)
                sv = srcb[j, sl]
                srcb[j, sl] = jnp.where(sv >= HALF, sv + (P - HALF), sv)
                dv = dstb[j, sl] - dst_base
                ok = (dv >= 0) & (dv < HALF)
                dlb[j, sl] = jnp.where(ok, dv, DUMMY)
            pltpu.async_copy(x_hbm.at[srcb.at[j]], rowsv, sem).wait()

            def scale_body(e, _):
                wv = plsc.load_gather(wb, [jnp.full((L,), j, jnp.int32),
                                           jnp.full((L,), e, jnp.int32)])
                for k in range(4):
                    q = pl.ds(k * L, L)
                    rowsv[e, q] = rowsv[e, q] * wv
                return 0

            lax.fori_loop(0, CH, scale_body, 0)
            pltpu.sync_copy(rowsv, acc.at[dlb.at[j]], add=True)
        return carry

    lax.fori_loop(0, NSUP, super_body, 0)
    plsc.subcore_barrier()
    pltpu.sync_copy(acc.at[pl.ds(s * RPS, RPS), :],
                    out_hbm.at[pl.ds(c * P + s * RPS, RPS), :])


_layer = pl.kernel(
    _layer_body,
    out_type=jax.ShapeDtypeStruct((2 * P, EMB), jnp.float32),
    mesh=plsc.VectorSubcoreMesh(core_axis_name="c", subcore_axis_name="s",
                                num_cores=NC, num_subcores=NS),
    scratch_types=[
        pltpu.VMEM_SHARED((P, EMB), jnp.float32),
        pltpu.VMEM((SUPER, CH), jnp.int32),
        pltpu.VMEM((SUPER, CH), jnp.int32),
        pltpu.VMEM((SUPER, CH), jnp.float32),
        pltpu.VMEM((SUPER, CH), jnp.int32),
        pltpu.VMEM((CH, EMB), jnp.float32),
        pltpu.SemaphoreType.DMA,
    ],
)


def _mean_body(a_ref, b_ref, c_ref, d_ref, o_ref):
    o_ref[...] = (a_ref[...] + b_ref[...] + c_ref[...] + d_ref[...]) * 0.25


def _mean4(x0, x1, x2, x3):
    rows = 2 * P
    blk = 1024
    return pl.pallas_call(
        _mean_body,
        out_shape=jax.ShapeDtypeStruct((rows, EMB), jnp.float32),
        grid=(rows // blk,),
        in_specs=[pl.BlockSpec((blk, EMB), lambda i: (i, 0))] * 4,
        out_specs=pl.BlockSpec((blk, EMB), lambda i: (i, 0)),
    )(x0, x1, x2, x3)


def kernel(edge_index, edge_weight, emb_weight):
    src = edge_index[0]
    dst = edge_index[1]
    pad = E_PAD - E
    src2 = jnp.concatenate([src, jnp.zeros((pad,), jnp.int32)]).reshape(ROWS, CH)
    dst2 = jnp.concatenate([dst, jnp.zeros((pad,), jnp.int32)]).reshape(ROWS, CH)
    w2 = jnp.concatenate(
        [edge_weight, jnp.zeros((pad,), jnp.float32)]).reshape(ROWS, CH)
    xp = jnp.zeros((2 * P, EMB), jnp.float32)
    xp = xp.at[:HALF].set(emb_weight[:HALF]).at[P:P + HALF].set(emb_weight[HALF:])
    zp = jnp.zeros((P, EMB), jnp.float32)

    outs = [xp]
    x = xp
    for _ in range(N_LAYERS):
        x = _layer(x, src2, dst2, w2, zp)
        outs.append(x)
    m = _mean4(*outs)
    return jnp.concatenate([m[:HALF], m[P:P + HALF]], axis=0)


# speedup vs baseline: 1.9114x; 1.9114x over previous
import jax
import jax.numpy as jnp
from jax import lax
from jax.experimental import pallas as pl
from jax.experimental.pallas import tpu as pltpu
from jax.experimental.pallas import tpu_sc as plsc

HALF = 25000
EMB = 64
E = 800000
N_LAYERS = 3
NC = 2
NS = 16
L = 16

P = 25088
RPS = P // NS
DUMMY = HALF
CH = 128
E_PAD = 819200
ROWS = E_PAD // CH
RPSUB = ROWS // NS
SUPER = 16
NSUP = RPSUB // SUPER


def _layer_body(x_hbm, src_hbm, dst_hbm, w_hbm, z_hbm, out_hbm,
                acc, srcb, dstb, wb, dlb, rowsv, sem):
    c = lax.axis_index("c")
    s = lax.axis_index("s")

    pltpu.sync_copy(z_hbm.at[pl.ds(s * RPS, RPS), :],
                    acc.at[pl.ds(s * RPS, RPS), :])
    plsc.subcore_barrier()

    dst_base = c * HALF
    row0 = s * RPSUB

    def super_body(it, carry):
        r0 = row0 + it * SUPER
        pltpu.sync_copy(src_hbm.at[pl.ds(r0, SUPER), :], srcb)
        pltpu.sync_copy(dst_hbm.at[pl.ds(r0, SUPER), :], dstb)
        pltpu.sync_copy(w_hbm.at[pl.ds(r0, SUPER), :], wb)
        for j in range(SUPER):
            for g in range(8):
                sl = pl.ds(g * L, L)
                sv = srcb[j, sl]
                srcb[j, sl] = jnp.where(sv >= HALF, sv + (P - HALF), sv)
                dv = dstb[j, sl] - dst_base
                ok = (dv >= 0) & (dv < HALF)
                dlb[j, sl] = jnp.where(ok, dv, DUMMY)
            pltpu.async_copy(x_hbm.at[srcb.at[j]], rowsv, sem).wait()

            def scale_body(g, _):
                w16 = wb[j, pl.ds(g * L, L)]
                for ee in range(L):
                    wsc = w16[ee]
                    e = g * L + ee
                    for k in range(4):
                        q = pl.ds(k * L, L)
                        rowsv[e, q] = rowsv[e, q] * wsc
                return 0

            lax.fori_loop(0, CH // L, scale_body, 0)
            pltpu.sync_copy(rowsv, acc.at[dlb.at[j]], add=True)
        return carry

    lax.fori_loop(0, NSUP, super_body, 0)
    plsc.subcore_barrier()
    pltpu.sync_copy(acc.at[pl.ds(s * RPS, RPS), :],
                    out_hbm.at[pl.ds(c * P + s * RPS, RPS), :])


_layer = pl.kernel(
    _layer_body,
    out_type=jax.ShapeDtypeStruct((2 * P, EMB), jnp.float32),
    mesh=plsc.VectorSubcoreMesh(core_axis_name="c", subcore_axis_name="s",
                                num_cores=NC, num_subcores=NS),
    compiler_params=pltpu.CompilerParams(use_tc_tiling_on_sc=False),
    scratch_types=[
        pltpu.VMEM_SHARED((P, EMB), jnp.float32),
        pltpu.VMEM((SUPER, CH), jnp.int32),
        pltpu.VMEM((SUPER, CH), jnp.int32),
        pltpu.VMEM((SUPER, CH), jnp.float32),
        pltpu.VMEM((SUPER, CH), jnp.int32),
        pltpu.VMEM((CH, EMB), jnp.float32),
        pltpu.SemaphoreType.DMA,
    ],
)


def _mean_body(a_ref, b_ref, c_ref, d_ref, o_ref):
    o_ref[...] = (a_ref[...] + b_ref[...] + c_ref[...] + d_ref[...]) * 0.25


def _mean4(x0, x1, x2, x3):
    rows = 2 * P
    blk = 1024
    return pl.pallas_call(
        _mean_body,
        out_shape=jax.ShapeDtypeStruct((rows, EMB), jnp.float32),
        grid=(rows // blk,),
        in_specs=[pl.BlockSpec((blk, EMB), lambda i: (i, 0))] * 4,
        out_specs=pl.BlockSpec((blk, EMB), lambda i: (i, 0)),
    )(x0, x1, x2, x3)


def kernel(edge_index, edge_weight, emb_weight):
    src = edge_index[0]
    dst = edge_index[1]
    pad = E_PAD - E
    src2 = jnp.concatenate([src, jnp.zeros((pad,), jnp.int32)]).reshape(ROWS, CH)
    dst2 = jnp.concatenate([dst, jnp.zeros((pad,), jnp.int32)]).reshape(ROWS, CH)
    w2 = jnp.concatenate(
        [edge_weight, jnp.zeros((pad,), jnp.float32)]).reshape(ROWS, CH)
    xp = jnp.zeros((2 * P, EMB), jnp.float32)
    xp = xp.at[:HALF].set(emb_weight[:HALF]).at[P:P + HALF].set(emb_weight[HALF:])
    zp = jnp.zeros((P, EMB), jnp.float32)

    outs = [xp]
    x = xp
    for _ in range(N_LAYERS):
        x = _layer(x, src2, dst2, w2, zp)
        outs.append(x)
    m = _mean4(*outs)
    return jnp.concatenate([m[:HALF], m[P:P + HALF]], axis=0)


# speedup vs baseline: 2.4019x; 1.2566x over previous
import jax
import jax.numpy as jnp
from jax import lax
from jax.experimental import pallas as pl
from jax.experimental.pallas import tpu as pltpu
from jax.experimental.pallas import tpu_sc as plsc

HALF = 25000
EMB = 64
E = 800000
N_LAYERS = 3
NC = 2
NS = 16
L = 16

P = 25088
RPS = P // NS
DUMMY = HALF
CH = 128
E_PAD = 819200
ROWS = E_PAD // CH
RPSUB = ROWS // NS
SUPER = 16
NSUP = RPSUB // SUPER


def _layer_body(x_hbm, src_hbm, dst_hbm, w_hbm, z_hbm, out_hbm,
                acc, srcb, dstb, wb, rows0, rows1, sem0, sem1):
    c = lax.axis_index("c")
    s = lax.axis_index("s")

    pltpu.sync_copy(z_hbm.at[pl.ds(s * RPS, RPS), :],
                    acc.at[pl.ds(s * RPS, RPS), :])
    plsc.subcore_barrier()

    dst_base = c * HALF

    def gissue(r, buf, sem):
        pltpu.async_copy(x_hbm.at[srcb.at[r]], buf, sem)

    def gwait(r, buf, sem):
        pltpu.make_async_copy(x_hbm.at[srcb.at[r]], buf, sem).wait()

    def process(r, buf):
        def scale_body(g, _):
            w16 = wb[r, pl.ds(g * L, L)]
            for ee in range(L):
                wsc = w16[ee]
                e = g * L + ee
                for k in range(4):
                    q = pl.ds(k * L, L)
                    buf[e, q] = buf[e, q] * wsc
            return 0

        lax.fori_loop(0, CH // L, scale_body, 0)
        pltpu.sync_copy(buf, acc.at[dstb.at[r]], add=True)

    def super_body(t, _):
        r0 = s * RPSUB + t * SUPER
        pltpu.sync_copy(src_hbm.at[pl.ds(r0, SUPER), :], srcb)
        pltpu.sync_copy(dst_hbm.at[pl.ds(r0, SUPER), :], dstb)
        pltpu.sync_copy(w_hbm.at[pl.ds(r0, SUPER), :], wb)

        def remap(r, _):
            for g in range(8):
                sl = pl.ds(g * L, L)
                sv = srcb[r, sl]
                srcb[r, sl] = jnp.where(sv >= HALF, sv + (P - HALF), sv)
                dv = dstb[r, sl] - dst_base
                ok = (dv >= 0) & (dv < HALF)
                dstb[r, sl] = jnp.where(ok, dv, DUMMY)
            return 0

        lax.fori_loop(0, SUPER, remap, 0)

        gissue(0, rows0, sem0)

        def pair(p, _):
            a = 2 * p
            b = a + 1
            gissue(b, rows1, sem1)
            gwait(a, rows0, sem0)
            process(a, rows0)
            gissue(jnp.minimum(a + 2, SUPER - 1), rows0, sem0)
            gwait(b, rows1, sem1)
            process(b, rows1)
            return 0

        lax.fori_loop(0, SUPER // 2, pair, 0)
        gwait(SUPER - 1, rows0, sem0)
        return 0

    lax.fori_loop(0, NSUP, super_body, 0)
    plsc.subcore_barrier()
    pltpu.sync_copy(acc.at[pl.ds(s * RPS, RPS), :],
                    out_hbm.at[pl.ds(c * P + s * RPS, RPS), :])


_layer = pl.kernel(
    _layer_body,
    out_type=jax.ShapeDtypeStruct((2 * P, EMB), jnp.float32),
    mesh=plsc.VectorSubcoreMesh(core_axis_name="c", subcore_axis_name="s",
                                num_cores=NC, num_subcores=NS),
    compiler_params=pltpu.CompilerParams(use_tc_tiling_on_sc=False),
    scratch_types=[
        pltpu.VMEM_SHARED((P, EMB), jnp.float32),
        pltpu.VMEM((SUPER, CH), jnp.int32),
        pltpu.VMEM((SUPER, CH), jnp.int32),
        pltpu.VMEM((SUPER, CH), jnp.float32),
        pltpu.VMEM((CH, EMB), jnp.float32),
        pltpu.VMEM((CH, EMB), jnp.float32),
        pltpu.SemaphoreType.DMA,
        pltpu.SemaphoreType.DMA,
    ],
)


def _mean_body(a_ref, b_ref, c_ref, d_ref, o_ref):
    o_ref[...] = (a_ref[...] + b_ref[...] + c_ref[...] + d_ref[...]) * 0.25


def _mean4(x0, x1, x2, x3):
    rows = 2 * P
    blk = 1024
    return pl.pallas_call(
        _mean_body,
        out_shape=jax.ShapeDtypeStruct((rows, EMB), jnp.float32),
        grid=(rows // blk,),
        in_specs=[pl.BlockSpec((blk, EMB), lambda i: (i, 0))] * 4,
        out_specs=pl.BlockSpec((blk, EMB), lambda i: (i, 0)),
    )(x0, x1, x2, x3)


def kernel(edge_index, edge_weight, emb_weight):
    src = edge_index[0]
    dst = edge_index[1]
    pad = E_PAD - E
    src2 = jnp.concatenate([src, jnp.zeros((pad,), jnp.int32)]).reshape(ROWS, CH)
    dst2 = jnp.concatenate([dst, jnp.zeros((pad,), jnp.int32)]).reshape(ROWS, CH)
    w2 = jnp.concatenate(
        [edge_weight, jnp.zeros((pad,), jnp.float32)]).reshape(ROWS, CH)
    xp = jnp.zeros((2 * P, EMB), jnp.float32)
    xp = xp.at[:HALF].set(emb_weight[:HALF]).at[P:P + HALF].set(emb_weight[HALF:])
    zp = jnp.zeros((P, EMB), jnp.float32)

    outs = [xp]
    x = xp
    for _ in range(N_LAYERS):
        x = _layer(x, src2, dst2, w2, zp)
        outs.append(x)
    m = _mean4(*outs)
    return jnp.concatenate([m[:HALF], m[P:P + HALF]], axis=0)
